# baseline (speedup 1.0000x reference)
import functools

import jax
import jax.numpy as jnp
from jax import lax
from jax.experimental import pallas as pl
from jax.experimental.pallas import tpu as pltpu
from jax.experimental.pallas import tpu_sc as plsc

N = 10000
E = 320000
D = 128
G = 64
C = 2

NC = 2
NS = 16
NW = NC * NS
NP = 10240
ROWS_PER_SUB = NP // NS
EPW = 10240
EP = EPW * NW
DCH = 128
DNCH = EPW // DCH

_mesh = plsc.VectorSubcoreMesh(core_axis_name="c", subcore_axis_name="s")


@functools.partial(
    pl.kernel,
    out_type=(
        jax.ShapeDtypeStruct((NP, D), jnp.float32),
        jax.ShapeDtypeStruct((NP, D), jnp.float32),
    ),
    mesh=_mesh,
    scratch_types=[
        pltpu.VMEM((DNCH, DCH), jnp.int32),
        pltpu.VMEM((DCH, D), jnp.float32),
        pltpu.VMEM_SHARED((NP, D), jnp.float32),
    ],
)
def _deg_kernel(dst_hbm, ones_hbm, zero_hbm, outa, outb, dstv, onesv, acc):
    cid = lax.axis_index("c")
    sid = lax.axis_index("s")
    wid = sid * NC + cid
    pltpu.sync_copy(dst_hbm.at[wid], dstv)
    pltpu.sync_copy(ones_hbm, onesv)
    pltpu.sync_copy(zero_hbm.at[pl.ds(sid * ROWS_PER_SUB, ROWS_PER_SUB)],
                    acc.at[pl.ds(sid * ROWS_PER_SUB, ROWS_PER_SUB)])
    plsc.subcore_barrier()

    def body(j, carry):
        pltpu.sync_copy(onesv, acc.at[dstv.at[j]], add=True)
        return carry

    lax.fori_loop(0, DNCH, body, 0)
    plsc.subcore_barrier()

    @pl.when(cid == 0)
    def _():
        pltpu.sync_copy(acc.at[pl.ds(sid * ROWS_PER_SUB, ROWS_PER_SUB)],
                        outa.at[pl.ds(sid * ROWS_PER_SUB, ROWS_PER_SUB)])

    @pl.when(cid == 1)
    def _():
        pltpu.sync_copy(acc.at[pl.ds(sid * ROWS_PER_SUB, ROWS_PER_SUB)],
                        outb.at[pl.ds(sid * ROWS_PER_SUB, ROWS_PER_SUB)])


CH = 64
SUP = 10
NBUF = 5
NCHT = EP // CH
TOTSUP = NCHT // SUP
F0 = 60
F1 = NCHT // NS - F0


@functools.partial(
    pl.kernel,
    out_type=(
        jax.ShapeDtypeStruct((NP, D), jnp.float32),
        jax.ShapeDtypeStruct((NP, D), jnp.float32),
    ),
    mesh=_mesh,
    scratch_types=[
        pltpu.VMEM((2, SUP, CH), jnp.int32),
        pltpu.VMEM((2, SUP, CH), jnp.int32),
        pltpu.VMEM((NBUF, CH, D), jnp.float32),
        pltpu.VMEM_SHARED((NP, D), jnp.float32),
        pltpu.SemaphoreType.DMA,
        pltpu.SemaphoreType.DMA,
        pltpu.SemaphoreType.DMA,
        pltpu.SemaphoreType.DMA,
        pltpu.SemaphoreType.DMA,
        pltpu.SemaphoreType.DMA,
        pltpu.SemaphoreType.DMA,
        pltpu.SemaphoreType.DMA,
        pltpu.SemaphoreType.DMA,
        pltpu.SemaphoreType.DMA,
        pltpu.SemaphoreType.DMA,
    ],
)
def _agg_kernel(src_hbm, dst_hbm, h2_hbm, zero_hbm, outa, outb,
                srcv, dstv, rows, acc, si,
                sg0, sg1, sg2, sg3, sg4, ss0, ss1, ss2, ss3, ss4):
    cid = lax.axis_index("c")
    sid = lax.axis_index("s")
    sg = (sg0, sg1, sg2, sg3, sg4)
    ss = (ss0, ss1, ss2, ss3, ss4)

    f_mine = jnp.where(cid == 0, F0, F1)
    grp_mine = jnp.where(cid == 0, F0 // NBUF, F1 // NBUF)
    nsup_mine = jnp.where(cid == 0, F0 // SUP, F1 // SUP)
    sbase = jnp.where(cid == 0, sid * (F0 // SUP),
                      NS * (F0 // SUP) + sid * (F1 // SUP))

    def supoff(j):
        s = j // SUP
        return s % 2, j % SUP

    def gath(j, b):
        sb, so = supoff(j)
        return pltpu.make_async_copy(h2_hbm.at[srcv.at[sb, so]],
                                     rows.at[b], sg[b])

    def scat_start(j, b):
        sb, so = supoff(j)
        pltpu.async_copy(rows.at[b], acc.at[dstv.at[sb, so]], ss[b],
                         add=True)

    def scat_wait(j, b):
        sb, so = supoff(j)
        pltpu.make_async_copy(rows.at[b], acc.at[dstv.at[sb, so]],
                              ss[b]).wait()

    pltpu.sync_copy(src_hbm.at[sbase], srcv.at[0])
    pltpu.sync_copy(dst_hbm.at[sbase], dstv.at[0])
    pltpu.sync_copy(zero_hbm.at[pl.ds(sid * ROWS_PER_SUB, ROWS_PER_SUB)],
                    acc.at[pl.ds(sid * ROWS_PER_SUB, ROWS_PER_SUB)])
    plsc.subcore_barrier()
    for k in range(3):
        pltpu.async_copy(h2_hbm.at[srcv.at[0, k]], rows.at[k], sg[k])

    def group(g, carry):
        gp = lax.rem(g, 2)
        for i in range(NBUF):
            j = g * NBUF + i
            @pl.when(j >= 2)
            def _():
                scat_wait(jnp.maximum(j - 2, 0), (i - 2) % NBUF)
            if i == 2:
                s_next = g // 2 + 1

                @pl.when((gp == 0) & (s_next < nsup_mine))
                def _():
                    pltpu.async_copy(src_hbm.at[sbase + s_next],
                                     srcv.at[lax.rem(s_next, 2)], si)
                    pltpu.async_copy(dst_hbm.at[sbase + s_next],
                                     dstv.at[lax.rem(s_next, 2)], si)

                @pl.when((gp == 1) & (s_next < nsup_mine))
                def _():
                    pltpu.make_async_copy(
                        src_hbm.at[sbase + s_next],
                        srcv.at[lax.rem(s_next, 2)], si).wait()
                    pltpu.make_async_copy(
                        dst_hbm.at[sbase + s_next],
                        dstv.at[lax.rem(s_next, 2)], si).wait()
            @pl.when(j + 3 < f_mine)
            def _():
                gath(jnp.minimum(j + 3, f_mine - 1), (i + 3) % NBUF).start()
            gath(j, i).wait()
            scat_start(j, i)
        return carry

    lax.fori_loop(0, grp_mine, group, 0)

    @pl.when(cid == 0)
    def _():
        scat_wait(F0 - 2, (F0 - 2) % NBUF)
        scat_wait(F0 - 1, (F0 - 1) % NBUF)

    @pl.when(cid == 1)
    def _():
        scat_wait(F1 - 2, (F1 - 2) % NBUF)
        scat_wait(F1 - 1, (F1 - 1) % NBUF)

    plsc.subcore_barrier()

    @pl.when(cid == 0)
    def _():
        pltpu.sync_copy(acc.at[pl.ds(sid * ROWS_PER_SUB, ROWS_PER_SUB)],
                        outa.at[pl.ds(sid * ROWS_PER_SUB, ROWS_PER_SUB)])

    @pl.when(cid == 1)
    def _():
        pltpu.sync_copy(acc.at[pl.ds(sid * ROWS_PER_SUB, ROWS_PER_SUB)],
                        outb.at[pl.ds(sid * ROWS_PER_SUB, ROWS_PER_SUB)])


def _valid_mask():
    rowid = lax.broadcasted_iota(jnp.int32, (NP, 1), 0)
    return (rowid < N).astype(jnp.float32)


def _bn_masked(r, valid, g, b):
    rv = r * valid
    m = jnp.sum(rv, axis=0, keepdims=True) * (1.0 / N)
    ex2 = jnp.sum(rv * rv, axis=0, keepdims=True) * (1.0 / N)
    v = ex2 - m * m
    return (r - m) * (1.0 / jnp.sqrt(v + 1e-5)) * g + b


def _mm(a, b):
    return lax.dot_general(a, b, (((1,), (0,)), ((), ())),
                           precision=lax.Precision.HIGHEST,
                           preferred_element_type=jnp.float32)


def _stage0_body(x_ref, dega_ref, degb_ref, g0_ref, b0_ref, w1_ref,
                 h2_ref, dinv_ref):
    valid = _valid_mask()
    deg = dega_ref[:, 0:1] + degb_ref[:, 0:1] + 1.0
    dinv = 1.0 / jnp.sqrt(deg)
    h = _bn_masked(x_ref[...], valid, g0_ref[...], b0_ref[...])
    h2 = _mm(h, w1_ref[...]) * dinv * valid
    h2_ref[...] = h2
    dinv_ref[...] = dinv


_stage0 = pl.pallas_call(
    _stage0_body,
    out_shape=(
        jax.ShapeDtypeStruct((NP, D), jnp.float32),
        jax.ShapeDtypeStruct((NP, 1), jnp.float32),
    ),
)


def _stagek_body(sa_ref, sb_ref, h2p_ref, dinv_ref, bias_ref, g_ref, b_ref,
                 w_ref, h2_ref):
    valid = _valid_mask()
    dinv = dinv_ref[...]
    p = dinv * (sa_ref[...] + sb_ref[...] + h2p_ref[...]) + bias_ref[...]
    r = jnp.maximum(p, 0.0)
    h = _bn_masked(r, valid, g_ref[...], b_ref[...])
    h2_ref[...] = _mm(h, w_ref[...]) * dinv * valid


_stagek = pl.pallas_call(
    _stagek_body,
    out_shape=jax.ShapeDtypeStruct((NP, D), jnp.float32),
)


def _stage3_body(sa_ref, sb_ref, h2p_ref, dinv_ref, bias_ref, g_ref, b_ref,
                 batch_ref, fcw1_ref, fcb1_ref, fcw2_ref, fcb2_ref, out_ref):
    valid = _valid_mask()
    dinv = dinv_ref[...]
    p = dinv * (sa_ref[...] + sb_ref[...] + h2p_ref[...]) + bias_ref[...]
    r = jnp.maximum(p, 0.0)
    h = _bn_masked(r, valid, g_ref[...], b_ref[...]) * valid
    gid = lax.broadcasted_iota(jnp.int32, (G, NP), 0)
    onehot = (batch_ref[...] == gid).astype(jnp.float32)
    sums = lax.dot_general(onehot, h, (((1,), (0,)), ((), ())),
                           precision=lax.Precision.HIGHEST,
                           preferred_element_type=jnp.float32)
    cnt = jnp.sum(onehot, axis=1, keepdims=True)
    pooled = sums / jnp.maximum(cnt, 1.0)
    z = jnp.maximum(_mm(pooled, fcw1_ref[...]) + fcb1_ref[...], 0.0)
    out_ref[...] = _mm(z, fcw2_ref[...]) + fcb2_ref[...]


_stage3 = pl.pallas_call(
    _stage3_body,
    out_shape=jax.ShapeDtypeStruct((G, C), jnp.float32),
)


def kernel(x, edge_index, batch, bn0_g, bn0_b, W1, b1, bn1_g, bn1_b, W2, b2,
           bn2_g, bn2_b, W3, b3, bn3_g, bn3_b, fcW1, fcb1, fcW2, fcb2):
    pad_e = jnp.full((EP - E,), N, jnp.int32)
    srcp = jnp.concatenate([edge_index[0], pad_e]).reshape(TOTSUP, SUP, CH)
    dstp = jnp.concatenate([edge_index[1], pad_e]).reshape(TOTSUP, SUP, CH)
    xp = jnp.pad(x, ((0, NP - N), (0, 0)))
    batchp = jnp.concatenate(
        [batch, jnp.full((NP - N,), G, jnp.int32)]).reshape(1, NP)

    ones128 = jnp.ones((DCH, D), jnp.float32)
    zeros128 = jnp.zeros((NP, D), jnp.float32)

    dega, degb = _deg_kernel(dstp.reshape(NW, DNCH, DCH), ones128, zeros128)
    h2, dinv = _stage0(xp, dega, degb,
                       bn0_g.reshape(1, D), bn0_b.reshape(1, D), W1)

    sa, sb = _agg_kernel(srcp, dstp, h2, zeros128)
    h2 = _stagek(sa, sb, h2, dinv, b1.reshape(1, D), bn1_g.reshape(1, D),
                 bn1_b.reshape(1, D), W2)

    sa, sb = _agg_kernel(srcp, dstp, h2, zeros128)
    h2 = _stagek(sa, sb, h2, dinv, b2.reshape(1, D), bn2_g.reshape(1, D),
                 bn2_b.reshape(1, D), W3)

    sa, sb = _agg_kernel(srcp, dstp, h2, zeros128)
    out = _stage3(sa, sb, h2, dinv, b3.reshape(1, D), bn3_g.reshape(1, D),
                  bn3_b.reshape(1, D), batchp, fcW1, fcb1.reshape(1, 64),
                  fcW2, fcb2.reshape(1, C))
    return out


# speedup vs baseline: 1.1034x; 1.1034x over previous
import functools

import jax
import jax.numpy as jnp
from jax import lax
from jax.experimental import pallas as pl
from jax.experimental.pallas import tpu as pltpu
from jax.experimental.pallas import tpu_sc as plsc

N = 10000
E = 320000
D = 128
G = 64
C = 2

NC = 2
NS = 16
NW = NC * NS
NP = 10240
ROWS_PER_SUB = NP // NS
EPW = 10240
EP = EPW * NW
DCH = 128
DNCH = EPW // DCH

_mesh = plsc.VectorSubcoreMesh(core_axis_name="c", subcore_axis_name="s")


@functools.partial(
    pl.kernel,
    out_type=(
        jax.ShapeDtypeStruct((NP, D), jnp.float32),
        jax.ShapeDtypeStruct((NP, D), jnp.float32),
    ),
    mesh=_mesh,
    scratch_types=[
        pltpu.VMEM((DNCH, DCH), jnp.int32),
        pltpu.VMEM((DCH, D), jnp.float32),
        pltpu.VMEM_SHARED((NP, D), jnp.float32),
    ],
)
def _deg_kernel(dst_hbm, ones_hbm, zero_hbm, outa, outb, dstv, onesv, acc):
    cid = lax.axis_index("c")
    sid = lax.axis_index("s")
    wid = sid * NC + cid
    pltpu.sync_copy(dst_hbm.at[wid], dstv)
    pltpu.sync_copy(ones_hbm, onesv)
    pltpu.sync_copy(zero_hbm.at[pl.ds(sid * ROWS_PER_SUB, ROWS_PER_SUB)],
                    acc.at[pl.ds(sid * ROWS_PER_SUB, ROWS_PER_SUB)])
    plsc.subcore_barrier()

    def body(j, carry):
        pltpu.sync_copy(onesv, acc.at[dstv.at[j]], add=True)
        return carry

    lax.fori_loop(0, DNCH, body, 0)
    plsc.subcore_barrier()

    @pl.when(cid == 0)
    def _():
        pltpu.sync_copy(acc.at[pl.ds(sid * ROWS_PER_SUB, ROWS_PER_SUB)],
                        outa.at[pl.ds(sid * ROWS_PER_SUB, ROWS_PER_SUB)])

    @pl.when(cid == 1)
    def _():
        pltpu.sync_copy(acc.at[pl.ds(sid * ROWS_PER_SUB, ROWS_PER_SUB)],
                        outb.at[pl.ds(sid * ROWS_PER_SUB, ROWS_PER_SUB)])


CH = 64
SUP = 10
NBUF = 5
NCHT = EP // CH
TOTSUP = NCHT // SUP
F0 = 260
F1 = NCHT // NS - F0


@functools.partial(
    pl.kernel,
    out_type=(
        jax.ShapeDtypeStruct((NP, D), jnp.float32),
        jax.ShapeDtypeStruct((NP, D), jnp.float32),
    ),
    mesh=_mesh,
    scratch_types=[
        pltpu.VMEM((2, SUP, CH), jnp.int32),
        pltpu.VMEM((2, SUP, CH), jnp.int32),
        pltpu.VMEM((NBUF, CH, D), jnp.float32),
        pltpu.VMEM_SHARED((NP, D), jnp.float32),
        pltpu.SemaphoreType.DMA,
        pltpu.SemaphoreType.DMA,
        pltpu.SemaphoreType.DMA,
        pltpu.SemaphoreType.DMA,
        pltpu.SemaphoreType.DMA,
        pltpu.SemaphoreType.DMA,
        pltpu.SemaphoreType.DMA,
        pltpu.SemaphoreType.DMA,
        pltpu.SemaphoreType.DMA,
        pltpu.SemaphoreType.DMA,
        pltpu.SemaphoreType.DMA,
    ],
)
def _agg_kernel(src_hbm, dst_hbm, h2_hbm, zero_hbm, outa, outb,
                srcv, dstv, rows, acc, si,
                sg0, sg1, sg2, sg3, sg4, ss0, ss1, ss2, ss3, ss4):
    cid = lax.axis_index("c")
    sid = lax.axis_index("s")
    sg = (sg0, sg1, sg2, sg3, sg4)
    ss = (ss0, ss1, ss2, ss3, ss4)

    f_mine = jnp.where(cid == 0, F0, F1)
    grp_mine = jnp.where(cid == 0, F0 // NBUF, F1 // NBUF)
    nsup_mine = jnp.where(cid == 0, F0 // SUP, F1 // SUP)
    sbase = jnp.where(cid == 0, sid * (F0 // SUP),
                      NS * (F0 // SUP) + sid * (F1 // SUP))

    def supoff(j):
        s = j // SUP
        return s % 2, j % SUP

    def gath(j, b):
        sb, so = supoff(j)
        return pltpu.make_async_copy(h2_hbm.at[srcv.at[sb, so]],
                                     rows.at[b], sg[b])

    def scat_start(j, b):
        sb, so = supoff(j)
        pltpu.async_copy(rows.at[b], acc.at[dstv.at[sb, so]], ss[b],
                         add=True)

    def scat_wait(j, b):
        sb, so = supoff(j)
        pltpu.make_async_copy(rows.at[b], acc.at[dstv.at[sb, so]],
                              ss[b]).wait()

    pltpu.sync_copy(src_hbm.at[sbase], srcv.at[0])
    pltpu.sync_copy(dst_hbm.at[sbase], dstv.at[0])
    pltpu.sync_copy(zero_hbm.at[pl.ds(sid * ROWS_PER_SUB, ROWS_PER_SUB)],
                    acc.at[pl.ds(sid * ROWS_PER_SUB, ROWS_PER_SUB)])
    plsc.subcore_barrier()
    for k in range(3):
        pltpu.async_copy(h2_hbm.at[srcv.at[0, k]], rows.at[k], sg[k])

    def group(g, carry):
        gp = lax.rem(g, 2)
        for i in range(NBUF):
            j = g * NBUF + i
            @pl.when(j >= 2)
            def _():
                scat_wait(jnp.maximum(j - 2, 0), (i - 2) % NBUF)
            if i == 2:
                s_next = g // 2 + 1

                @pl.when((gp == 0) & (s_next < nsup_mine))
                def _():
                    pltpu.async_copy(src_hbm.at[sbase + s_next],
                                     srcv.at[lax.rem(s_next, 2)], si)
                    pltpu.async_copy(dst_hbm.at[sbase + s_next],
                                     dstv.at[lax.rem(s_next, 2)], si)

                @pl.when((gp == 1) & (s_next < nsup_mine))
                def _():
                    pltpu.make_async_copy(
                        src_hbm.at[sbase + s_next],
                        srcv.at[lax.rem(s_next, 2)], si).wait()
                    pltpu.make_async_copy(
                        dst_hbm.at[sbase + s_next],
                        dstv.at[lax.rem(s_next, 2)], si).wait()
            @pl.when(j + 3 < f_mine)
            def _():
                gath(jnp.minimum(j + 3, f_mine - 1), (i + 3) % NBUF).start()
            gath(j, i).wait()
            scat_start(j, i)
        return carry

    lax.fori_loop(0, grp_mine, group, 0)

    @pl.when(cid == 0)
    def _():
        scat_wait(F0 - 2, (F0 - 2) % NBUF)
        scat_wait(F0 - 1, (F0 - 1) % NBUF)

    @pl.when(cid == 1)
    def _():
        scat_wait(F1 - 2, (F1 - 2) % NBUF)
        scat_wait(F1 - 1, (F1 - 1) % NBUF)

    plsc.subcore_barrier()

    @pl.when(cid == 0)
    def _():
        pltpu.sync_copy(acc.at[pl.ds(sid * ROWS_PER_SUB, ROWS_PER_SUB)],
                        outa.at[pl.ds(sid * ROWS_PER_SUB, ROWS_PER_SUB)])

    @pl.when(cid == 1)
    def _():
        pltpu.sync_copy(acc.at[pl.ds(sid * ROWS_PER_SUB, ROWS_PER_SUB)],
                        outb.at[pl.ds(sid * ROWS_PER_SUB, ROWS_PER_SUB)])


def _valid_mask():
    rowid = lax.broadcasted_iota(jnp.int32, (NP, 1), 0)
    return (rowid < N).astype(jnp.float32)


def _bn_masked(r, valid, g, b):
    rv = r * valid
    m = jnp.sum(rv, axis=0, keepdims=True) * (1.0 / N)
    ex2 = jnp.sum(rv * rv, axis=0, keepdims=True) * (1.0 / N)
    v = ex2 - m * m
    return (r - m) * (1.0 / jnp.sqrt(v + 1e-5)) * g + b


def _mm(a, b):
    return lax.dot_general(a, b, (((1,), (0,)), ((), ())),
                           precision=lax.Precision.HIGHEST,
                           preferred_element_type=jnp.float32)


def _stage0_body(x_ref, dega_ref, degb_ref, g0_ref, b0_ref, w1_ref,
                 h2_ref, dinv_ref):
    valid = _valid_mask()
    deg = dega_ref[:, 0:1] + degb_ref[:, 0:1] + 1.0
    dinv = 1.0 / jnp.sqrt(deg)
    h = _bn_masked(x_ref[...], valid, g0_ref[...], b0_ref[...])
    h2 = _mm(h, w1_ref[...]) * dinv * valid
    h2_ref[...] = h2
    dinv_ref[...] = dinv


_stage0 = pl.pallas_call(
    _stage0_body,
    out_shape=(
        jax.ShapeDtypeStruct((NP, D), jnp.float32),
        jax.ShapeDtypeStruct((NP, 1), jnp.float32),
    ),
)


def _stagek_body(sa_ref, sb_ref, h2p_ref, dinv_ref, bias_ref, g_ref, b_ref,
                 w_ref, h2_ref):
    valid = _valid_mask()
    dinv = dinv_ref[...]
    p = dinv * (sa_ref[...] + sb_ref[...] + h2p_ref[...]) + bias_ref[...]
    r = jnp.maximum(p, 0.0)
    h = _bn_masked(r, valid, g_ref[...], b_ref[...])
    h2_ref[...] = _mm(h, w_ref[...]) * dinv * valid


_stagek = pl.pallas_call(
    _stagek_body,
    out_shape=jax.ShapeDtypeStruct((NP, D), jnp.float32),
)


def _stage3_body(sa_ref, sb_ref, h2p_ref, dinv_ref, bias_ref, g_ref, b_ref,
                 batch_ref, fcw1_ref, fcb1_ref, fcw2_ref, fcb2_ref, out_ref):
    valid = _valid_mask()
    dinv = dinv_ref[...]
    p = dinv * (sa_ref[...] + sb_ref[...] + h2p_ref[...]) + bias_ref[...]
    r = jnp.maximum(p, 0.0)
    h = _bn_masked(r, valid, g_ref[...], b_ref[...]) * valid
    gid = lax.broadcasted_iota(jnp.int32, (G, NP), 0)
    onehot = (batch_ref[...] == gid).astype(jnp.float32)
    sums = lax.dot_general(onehot, h, (((1,), (0,)), ((), ())),
                           precision=lax.Precision.HIGHEST,
                           preferred_element_type=jnp.float32)
    cnt = jnp.sum(onehot, axis=1, keepdims=True)
    pooled = sums / jnp.maximum(cnt, 1.0)
    z = jnp.maximum(_mm(pooled, fcw1_ref[...]) + fcb1_ref[...], 0.0)
    out_ref[...] = _mm(z, fcw2_ref[...]) + fcb2_ref[...]


_stage3 = pl.pallas_call(
    _stage3_body,
    out_shape=jax.ShapeDtypeStruct((G, C), jnp.float32),
)


def kernel(x, edge_index, batch, bn0_g, bn0_b, W1, b1, bn1_g, bn1_b, W2, b2,
           bn2_g, bn2_b, W3, b3, bn3_g, bn3_b, fcW1, fcb1, fcW2, fcb2):
    pad_e = jnp.full((EP - E,), N, jnp.int32)
    srcp = jnp.concatenate([edge_index[0], pad_e]).reshape(TOTSUP, SUP, CH)
    dstp = jnp.concatenate([edge_index[1], pad_e]).reshape(TOTSUP, SUP, CH)
    xp = jnp.pad(x, ((0, NP - N), (0, 0)))
    batchp = jnp.concatenate(
        [batch, jnp.full((NP - N,), G, jnp.int32)]).reshape(1, NP)

    ones128 = jnp.ones((DCH, D), jnp.float32)
    zeros128 = jnp.zeros((NP, D), jnp.float32)

    dega, degb = _deg_kernel(dstp.reshape(NW, DNCH, DCH), ones128, zeros128)
    h2, dinv = _stage0(xp, dega, degb,
                       bn0_g.reshape(1, D), bn0_b.reshape(1, D), W1)

    sa, sb = _agg_kernel(srcp, dstp, h2, zeros128)
    h2 = _stagek(sa, sb, h2, dinv, b1.reshape(1, D), bn1_g.reshape(1, D),
                 bn1_b.reshape(1, D), W2)

    sa, sb = _agg_kernel(srcp, dstp, h2, zeros128)
    h2 = _stagek(sa, sb, h2, dinv, b2.reshape(1, D), bn2_g.reshape(1, D),
                 bn2_b.reshape(1, D), W3)

    sa, sb = _agg_kernel(srcp, dstp, h2, zeros128)
    out = _stage3(sa, sb, h2, dinv, b3.reshape(1, D), bn3_g.reshape(1, D),
                  bn3_b.reshape(1, D), batchp, fcW1, fcb1.reshape(1, 64),
                  fcW2, fcb2.reshape(1, C))
    return out
